# baseline (device time: 976255 ns/iter reference)
import jax
import jax.numpy as jnp
from jax import lax
from jax.experimental import pallas as pl
from jax.experimental.pallas import tpu as pltpu

C = 32
LC = 16


def kernel(x):
    m, n = x.shape
    half = m // 2
    ch = half // C
    lch = m // LC

    def body(x_hbm, out_hbm, comm, own_buf, oisems, oosems,
             ybuf, yisems, yosems, xbuf, xisems, xosems,
             ysend, yrecv, xsend, xrecv):
        my_x = lax.axis_index("x")
        my_y = lax.axis_index("y")
        other_y = 1 - my_y
        other_x = 1 - my_x

        barrier = pltpu.get_barrier_semaphore()
        for dev in [(my_x, other_y), (other_x, my_y)]:
            pl.semaphore_signal(
                barrier, inc=1,
                device_id=dev, device_id_type=pl.DeviceIdType.MESH,
            )
        pl.semaphore_wait(barrier, 2)

        def make_stager(buf, isems, osems, n_chunks, src_fn, dst_fn):
            stores = [None] * n_chunks
            idx = [0]

            def push():
                c = idx[0]
                idx[0] += 1
                slot = c % 2
                if c >= 2:
                    stores[c - 2].wait()
                ld = pltpu.make_async_copy(src_fn(c), buf.at[slot], isems.at[slot])
                ld.start()
                ld.wait()
                st = pltpu.make_async_copy(buf.at[slot], dst_fn(c), osems.at[slot])
                st.start()
                stores[c] = st

            def drain():
                for k in (n_chunks - 2, n_chunks - 1):
                    stores[k].wait()

            return push, drain

        push_own, drain_own = make_stager(
            own_buf, oisems, oosems, LC,
            lambda c: x_hbm.at[pl.ds(c * lch, lch), :],
            lambda c: out_hbm.at[pl.ds(my_y * m + c * lch, lch), :],
        )
        push_y, drain_y = make_stager(
            ybuf, yisems, yosems, C,
            lambda c: comm.at[pl.ds(my_x * half + c * ch, ch), :],
            lambda c: out_hbm.at[pl.ds(other_y * m + my_x * half + c * ch, ch), :],
        )
        push_x, drain_x = make_stager(
            xbuf, xisems, xosems, C,
            lambda c: comm.at[pl.ds(other_x * half + c * ch, ch), :],
            lambda c: out_hbm.at[pl.ds(other_y * m + other_x * half + c * ch, ch), :],
        )

        ydmas = []
        for c in range(C):
            off = my_x * half + c * ch
            ydma = pltpu.make_async_remote_copy(
                src_ref=x_hbm.at[pl.ds(off, ch), :],
                dst_ref=comm.at[pl.ds(off, ch), :],
                send_sem=ysend.at[c],
                recv_sem=yrecv.at[c],
                device_id=(my_x, other_y),
                device_id_type=pl.DeviceIdType.MESH,
            )
            ydma.start()
            ydmas.append(ydma)

        xdmas = []
        for c in range(C):
            ydmas[c].wait_recv()
            off = my_x * half + c * ch
            xdma = pltpu.make_async_remote_copy(
                src_ref=comm.at[pl.ds(off, ch), :],
                dst_ref=comm.at[pl.ds(off, ch), :],
                send_sem=xsend.at[c],
                recv_sem=xrecv.at[c],
                device_id=(other_x, my_y),
                device_id_type=pl.DeviceIdType.MESH,
            )
            xdma.start()
            xdmas.append(xdma)
            if c % 2 == 0:
                push_own()
            push_y()

        for c in range(C):
            xdmas[c].wait_recv()
            push_x()

        drain_own()
        drain_y()
        drain_x()
        for c in range(C):
            ydmas[c].wait_send()
            xdmas[c].wait_send()

    out, _ = pl.pallas_call(
        body,
        out_shape=[
            jax.ShapeDtypeStruct((2 * m, n), x.dtype),
            jax.ShapeDtypeStruct((m, n), x.dtype),
        ],
        in_specs=[pl.BlockSpec(memory_space=pl.ANY)],
        out_specs=[
            pl.BlockSpec(memory_space=pl.ANY),
            pl.BlockSpec(memory_space=pl.ANY),
        ],
        scratch_shapes=[
            pltpu.VMEM((2, m // LC, n), jnp.float32),
            pltpu.SemaphoreType.DMA((2,)),
            pltpu.SemaphoreType.DMA((2,)),
            pltpu.VMEM((2, half // C, n), jnp.float32),
            pltpu.SemaphoreType.DMA((2,)),
            pltpu.SemaphoreType.DMA((2,)),
            pltpu.VMEM((2, half // C, n), jnp.float32),
            pltpu.SemaphoreType.DMA((2,)),
            pltpu.SemaphoreType.DMA((2,)),
            pltpu.SemaphoreType.DMA((C,)),
            pltpu.SemaphoreType.DMA((C,)),
            pltpu.SemaphoreType.DMA((C,)),
            pltpu.SemaphoreType.DMA((C,)),
        ],
        compiler_params=pltpu.CompilerParams(collective_id=0),
    )(x)
    return out
